# baseline (speedup 1.0000x reference)
import functools

import jax
import jax.numpy as jnp
from jax import lax
from jax.experimental import pallas as pl
from jax.experimental.pallas import tpu as pltpu
from jax.experimental.pallas import tpu_sc as plsc

NUM_CORES = 2
NUM_SUBCORES = 16
LANES = 16
NUM_WORKERS = NUM_CORES * NUM_SUBCORES

BATCH = 16384
DIM = 32
CHUNK = BATCH // NUM_WORKERS
R = 8
NPAIR = CHUNK // (2 * R)


def _body(user_hbm, item_hbm, ue_hbm, ie_hbm, ub_hbm, ib_hbm, off_hbm,
          out_hbm, uidx_v, iidx_v, blk_v, ub_v, ib_v, out_v, off_v, sem):
    wid = lax.axis_index("s") * NUM_CORES + lax.axis_index("c")
    base = wid * CHUNK

    for j in range(CHUNK // 128):
        pltpu.sync_copy(user_hbm.at[pl.ds(base + j * 128, 128)], uidx_v.at[j])
        pltpu.sync_copy(item_hbm.at[pl.ds(base + j * 128, 128)], iidx_v.at[j])
    pltpu.sync_copy(off_hbm, off_v.at[pl.ds(0, 1)])
    off = off_v[pl.ds(0, LANES)][0]

    iv = lax.iota(jnp.int32, LANES)
    ivu = iv % R
    ivi = ivu + R
    himask = iv >= R

    def round_acc(p, half):
        rr = 2 * p + half
        row = rr // (128 // R)
        col0 = (rr % (128 // R)) * R
        rowvec = jnp.full((LANES,), row, jnp.int32)
        ru = plsc.load_gather(uidx_v, (rowvec, col0 + ivu))
        ri = plsc.load_gather(iidx_v, (rowvec, col0 + ivu))
        lane_u = ru % 128
        lane_i = ri % 128
        copies = []
        for i in range(R):
            su = pl.multiple_of((ru[i] // 128) * 128, 128)
            si = pl.multiple_of((ri[i] // 128) * 128, 128)
            for t in range(DIM // 8):
                ts = pl.ds(t * 8, 8)
                copies.append(pltpu.async_copy(
                    ue_hbm.at[ts, pl.ds(su, 128)], blk_v.at[i].at[ts], sem))
                copies.append(pltpu.async_copy(
                    ie_hbm.at[ts, pl.ds(si, 128)], blk_v.at[R + i].at[ts], sem))
        bu = pltpu.async_copy(
            ub_hbm.at[uidx_v.at[row].at[pl.ds(col0, R)]], ub_v, sem)
        bi = pltpu.async_copy(
            ib_hbm.at[iidx_v.at[row].at[pl.ds(col0, R)]], ib_v, sem)
        for c in copies:
            c.wait()
        bu.wait()
        bi.wait()
        acc = (plsc.load_gather(ub_v, (ivu,)) + plsc.load_gather(ib_v, (ivu,))
               + off)
        for d in range(DIM):
            dvec = jnp.full((LANES,), d, jnp.int32)
            gu = plsc.load_gather(blk_v, (ivu, dvec, lane_u))
            gi = plsc.load_gather(blk_v, (ivi, dvec, lane_i))
            acc = acc + gu * gi
        return acc

    def pair(p, carry):
        acc_a = round_acc(p, 0)
        acc_b = round_acc(p, 1)
        acc = jnp.where(himask, acc_b, acc_a)
        out_v[pl.ds(p * LANES, LANES)] = 5.5 / (1.0 + jnp.exp(-acc))
        return carry

    lax.fori_loop(0, NPAIR, pair, 0)
    pltpu.sync_copy(out_v, out_hbm.at[pl.ds(base, CHUNK)])


@jax.jit
def kernel(user, item, user_emb, item_emb, user_bias, item_bias, offset):
    run = functools.partial(
        pl.kernel,
        out_type=jax.ShapeDtypeStruct((BATCH,), jnp.float32),
        mesh=plsc.VectorSubcoreMesh(core_axis_name="c", subcore_axis_name="s"),
        scratch_types=[
            pltpu.VMEM((CHUNK // 128, 128), jnp.int32),
            pltpu.VMEM((CHUNK // 128, 128), jnp.int32),
            pltpu.VMEM((2 * R, DIM, 128), jnp.float32),
            pltpu.VMEM((R,), jnp.float32),
            pltpu.VMEM((R,), jnp.float32),
            pltpu.VMEM((CHUNK,), jnp.float32),
            pltpu.VMEM((LANES,), jnp.float32),
            pltpu.SemaphoreType.DMA,
        ],
        compiler_params=pltpu.CompilerParams(
            needs_layout_passes=False, use_tc_tiling_on_sc=True),
    )(_body)
    return run(user.astype(jnp.int32), item.astype(jnp.int32),
               user_emb.T, item_emb.T, user_bias, item_bias, offset)


# speedup vs baseline: 1.0058x; 1.0058x over previous
import functools

import jax
import jax.numpy as jnp
from jax import lax
from jax.experimental import pallas as pl
from jax.experimental.pallas import tpu as pltpu
from jax.experimental.pallas import tpu_sc as plsc

NUM_CORES = 2
NUM_SUBCORES = 16
LANES = 16
NUM_WORKERS = NUM_CORES * NUM_SUBCORES

BATCH = 16384
DIM = 32
CHUNK = BATCH // NUM_WORKERS
R = 8
NROUNDS = CHUNK // R
NTASKS = 2 * NROUNDS


def _body(user_hbm, item_hbm, ue_hbm, ie_hbm, ub_hbm, ib_hbm, off_hbm,
          out_hbm, uidx_v, iidx_v, s0, s1, s2, ub_v, ib_v, out_v, off_v,
          d0, d1, d2, db):
    slots = (s0, s1, s2)
    sems = (d0, d1, d2)
    wid = lax.axis_index("s") * NUM_CORES + lax.axis_index("c")
    base = wid * CHUNK

    for j in range(CHUNK // 128):
        pltpu.sync_copy(user_hbm.at[pl.ds(base + j * 128, 128)], uidx_v.at[j])
        pltpu.sync_copy(item_hbm.at[pl.ds(base + j * 128, 128)], iidx_v.at[j])
    pltpu.sync_copy(off_hbm, off_v.at[pl.ds(0, 1)])
    off = off_v[pl.ds(0, LANES)][0]

    iv = lax.iota(jnp.int32, LANES)
    ivu = iv % R
    lomask = iv < R

    def round_idx(rr, which):
        row = rr // (128 // R)
        col0 = (rr % (128 // R)) * R
        rowvec = ivu * 0 + row
        src = uidx_v if which == 0 else iidx_v
        return plsc.load_gather(src, (rowvec, col0 + ivu))

    def fire(rr, which, s):
        rv = round_idx(rr, which)
        table = ue_hbm if which == 0 else ie_hbm
        for i in range(R):
            st = pl.multiple_of((rv[i] // 128) * 128, 128)
            pltpu.async_copy(table.at[:, pl.ds(st, 128)],
                             slots[s].at[i], sems[s])

    def fire_dyn(rr, which, s):
        @pl.when(rr < NROUNDS)
        def _():
            fire(rr, which, s)

    def fire_bias(rr):
        row = rr // (128 // R)
        col0 = (rr % (128 // R)) * R
        p8 = (rr % 2) * R
        pltpu.async_copy(ub_hbm.at[uidx_v.at[row].at[pl.ds(col0, R)]],
                         ub_v.at[pl.ds(p8, R)], db)
        pltpu.async_copy(ib_hbm.at[iidx_v.at[row].at[pl.ds(col0, R)]],
                         ib_v.at[pl.ds(p8, R)], db)

    def fire_bias_dyn(rr):
        @pl.when(rr < NROUNDS)
        def _():
            fire_bias(rr)

    def wait_task(s):
        for i in range(R):
            pltpu.make_async_copy(ue_hbm.at[:, pl.ds(0, 128)],
                                  slots[s].at[i], sems[s]).wait()

    def wait_bias():
        pltpu.make_async_copy(ub_hbm.at[pl.ds(0, R)],
                              ub_v.at[pl.ds(0, R)], db).wait()
        pltpu.make_async_copy(ib_hbm.at[pl.ds(0, R)],
                              ib_v.at[pl.ds(0, R)], db).wait()

    def compute(rr, su, si):
        ru = round_idx(rr, 0)
        ri = round_idx(rr, 1)
        lane_u = ru % 128
        lane_i = ri % 128
        p8 = (rr % 2) * R
        wait_task(su)
        wait_task(si)
        wait_bias()
        acc = (plsc.load_gather(ub_v, (p8 + ivu,))
               + plsc.load_gather(ib_v, (p8 + ivu,)) + off)
        for d in range(DIM):
            dvec = jnp.full((LANES,), d, jnp.int32)
            gu = plsc.load_gather(slots[su], (ivu, dvec, lane_u))
            gi = plsc.load_gather(slots[si], (ivu, dvec, lane_i))
            acc = acc + gu * gi
        vals = 5.5 / (1.0 + jnp.exp(-acc))
        plsc.store_scatter(out_v, (rr * R + ivu,), vals, mask=lomask)

    fire_bias(0)
    fire_bias(1)
    fire(0, 0, 0)
    fire(0, 1, 1)
    fire(1, 0, 2)
    compute(0, 0, 1)
    fire(1, 1, 0)
    fire(2, 0, 1)
    fire_bias(2)

    def body(k, carry):
        r1 = 1 + 3 * k
        compute(r1, 2, 0)
        fire_dyn(r1 + 1, 1, 2)
        fire_dyn(r1 + 2, 0, 0)
        fire_bias_dyn(r1 + 2)
        r2 = r1 + 1
        compute(r2, 1, 2)
        fire_dyn(r2 + 1, 1, 1)
        fire_dyn(r2 + 2, 0, 2)
        fire_bias_dyn(r2 + 2)
        r3 = r2 + 1
        compute(r3, 0, 1)
        fire_dyn(r3 + 1, 1, 0)
        fire_dyn(r3 + 2, 0, 1)
        fire_bias_dyn(r3 + 2)
        return carry

    lax.fori_loop(0, (NROUNDS - 1) // 3, body, 0)
    pltpu.sync_copy(out_v, out_hbm.at[pl.ds(base, CHUNK)])


@jax.jit
def kernel(user, item, user_emb, item_emb, user_bias, item_bias, offset):
    run = functools.partial(
        pl.kernel,
        out_type=jax.ShapeDtypeStruct((BATCH,), jnp.float32),
        mesh=plsc.VectorSubcoreMesh(core_axis_name="c", subcore_axis_name="s"),
        scratch_types=[
            pltpu.VMEM((CHUNK // 128, 128), jnp.int32),
            pltpu.VMEM((CHUNK // 128, 128), jnp.int32),
            pltpu.VMEM((R, DIM, 128), jnp.float32),
            pltpu.VMEM((R, DIM, 128), jnp.float32),
            pltpu.VMEM((R, DIM, 128), jnp.float32),
            pltpu.VMEM((2 * R,), jnp.float32),
            pltpu.VMEM((2 * R,), jnp.float32),
            pltpu.VMEM((CHUNK,), jnp.float32),
            pltpu.VMEM((LANES,), jnp.float32),
            pltpu.SemaphoreType.DMA,
            pltpu.SemaphoreType.DMA,
            pltpu.SemaphoreType.DMA,
            pltpu.SemaphoreType.DMA,
        ],
        compiler_params=pltpu.CompilerParams(
            needs_layout_passes=False, use_tc_tiling_on_sc=True),
    )(_body)
    return run(user.astype(jnp.int32), item.astype(jnp.int32),
               user_emb.T, item_emb.T, user_bias, item_bias, offset)


# speedup vs baseline: 1.0290x; 1.0232x over previous
import functools

import jax
import jax.numpy as jnp
from jax import lax
from jax.experimental import pallas as pl
from jax.experimental.pallas import tpu as pltpu
from jax.experimental.pallas import tpu_sc as plsc

NUM_CORES = 2
NUM_SUBCORES = 16
LANES = 16
NUM_WORKERS = NUM_CORES * NUM_SUBCORES

BATCH = 16384
DIM = 32
CHUNK = BATCH // NUM_WORKERS
NGROUPS = CHUNK // LANES


def _body(user_hbm, item_hbm, ue_hbm, ie_hbm, ub_hbm, ib_hbm, off_hbm,
          out_hbm, uidx_v, iidx_v, blk_v, uc_v, bias_v, out_v, off_v, sem):
    wid = lax.axis_index("s") * NUM_CORES + lax.axis_index("c")
    base = wid * CHUNK

    pltpu.sync_copy(user_hbm.at[pl.ds(base, CHUNK)], uidx_v)
    pltpu.sync_copy(item_hbm.at[pl.ds(base, CHUNK)], iidx_v)
    pltpu.sync_copy(off_hbm, off_v.at[pl.ds(0, 1)])
    off = off_v[pl.ds(0, LANES)][0]

    ivec = lax.iota(jnp.int32, LANES)

    def fetch_blocks(table_hbm, r):
        copies = []
        for i in range(LANES):
            start = pl.multiple_of((r[i] // 128) * 128, 128)
            copies.append(pltpu.async_copy(
                table_hbm.at[:, pl.ds(start, 128)], blk_v.at[i], sem))
        return copies

    def fetch_bias(bias_hbm, r, half):
        copies = []
        for i in range(LANES):
            start = pl.multiple_of((r[i] // 128) * 128, 128)
            copies.append(pltpu.async_copy(
                bias_hbm.at[pl.ds(start, 128)], bias_v.at[half * LANES + i],
                sem))
        return copies

    def group(g, carry):
        gbase = g * LANES
        ru = uidx_v[pl.ds(gbase, LANES)]
        ri = iidx_v[pl.ds(gbase, LANES)]
        lane_u = ru % 128
        lane_i = ri % 128

        copies = fetch_blocks(ue_hbm, ru)
        copies += fetch_bias(ub_hbm, ru, 0)
        copies += fetch_bias(ib_hbm, ri, 1)
        for c in copies:
            c.wait()
        for d in range(DIM):
            dvec = jnp.full((LANES,), d, jnp.int32)
            uc_v[d, pl.ds(0, LANES)] = plsc.load_gather(
                blk_v, (ivec, dvec, lane_u))
        ub = plsc.load_gather(bias_v, (ivec, lane_u))
        ib = plsc.load_gather(bias_v, (ivec + LANES, lane_i))

        copies = fetch_blocks(ie_hbm, ri)
        for c in copies:
            c.wait()
        acc = ub + ib + off
        for d in range(DIM):
            dvec = jnp.full((LANES,), d, jnp.int32)
            acc = acc + uc_v[d, pl.ds(0, LANES)] * plsc.load_gather(
                blk_v, (ivec, dvec, lane_i))

        out_v[pl.ds(gbase, LANES)] = 5.5 / (1.0 + jnp.exp(-acc))
        return carry

    lax.fori_loop(0, NGROUPS, group, 0)
    pltpu.sync_copy(out_v, out_hbm.at[pl.ds(base, CHUNK)])


@jax.jit
def kernel(user, item, user_emb, item_emb, user_bias, item_bias, offset):
    run = functools.partial(
        pl.kernel,
        out_type=jax.ShapeDtypeStruct((BATCH,), jnp.float32),
        mesh=plsc.VectorSubcoreMesh(core_axis_name="c", subcore_axis_name="s"),
        scratch_types=[
            pltpu.VMEM((CHUNK,), jnp.int32),
            pltpu.VMEM((CHUNK,), jnp.int32),
            pltpu.VMEM((LANES, DIM, 128), jnp.float32),
            pltpu.VMEM((DIM, LANES), jnp.float32),
            pltpu.VMEM((2 * LANES, 128), jnp.float32),
            pltpu.VMEM((CHUNK,), jnp.float32),
            pltpu.VMEM((LANES,), jnp.float32),
            pltpu.SemaphoreType.DMA,
        ],
        compiler_params=pltpu.CompilerParams(
            needs_layout_passes=False, use_tc_tiling_on_sc=True),
    )(_body)
    return run(user.astype(jnp.int32), item.astype(jnp.int32),
               user_emb.T, item_emb.T, user_bias, item_bias, offset)


# speedup vs baseline: 1.0547x; 1.0249x over previous
import functools

import jax
import jax.numpy as jnp
from jax import lax
from jax.experimental import pallas as pl
from jax.experimental.pallas import tpu as pltpu
from jax.experimental.pallas import tpu_sc as plsc

NUM_CORES = 2
NUM_SUBCORES = 16
LANES = 16
NUM_WORKERS = NUM_CORES * NUM_SUBCORES

BATCH = 16384
DIM = 32
CHUNK = BATCH // NUM_WORKERS
NGROUPS = CHUNK // LANES


def _body(user_hbm, item_hbm, ue_hbm, ie_hbm, ub_hbm, ib_hbm, off_hbm,
          out_hbm, uidx_v, iidx_v, blk_v, uc_v, bias_v, out_v, off_v, sem):
    wid = lax.axis_index("s") * NUM_CORES + lax.axis_index("c")
    base = wid * CHUNK

    pltpu.sync_copy(user_hbm.at[pl.ds(base, CHUNK)], uidx_v)
    pltpu.sync_copy(item_hbm.at[pl.ds(base, CHUNK)], iidx_v)
    pltpu.sync_copy(off_hbm, off_v.at[pl.ds(0, 1)])
    off = off_v[pl.ds(0, LANES)][0]

    ivec = lax.iota(jnp.int32, LANES)

    def fetch_blocks(table_hbm, r):
        copies = []
        for i in range(LANES):
            start = pl.multiple_of((r[i] // 128) * 128, 128)
            copies.append(pltpu.async_copy(
                table_hbm.at[:, pl.ds(start, 128)], blk_v.at[i], sem))
        return copies

    def fetch_bias(bias_hbm, idx_ref, g, half):
        return [pltpu.async_copy(
            bias_hbm.at[idx_ref.at[pl.ds(g * LANES, LANES)]],
            bias_v.at[pl.ds(half * LANES, LANES)], sem)]

    def group(g, carry):
        gbase = g * LANES
        ru = uidx_v[pl.ds(gbase, LANES)]
        ri = iidx_v[pl.ds(gbase, LANES)]
        lane_u = ru % 128
        lane_i = ri % 128

        copies = fetch_blocks(ue_hbm, ru)
        copies += fetch_bias(ub_hbm, uidx_v, g, 0)
        copies += fetch_bias(ib_hbm, iidx_v, g, 1)
        for c in copies:
            c.wait()
        for d in range(DIM):
            dvec = jnp.full((LANES,), d, jnp.int32)
            uc_v[d, pl.ds(0, LANES)] = plsc.load_gather(
                blk_v, (ivec, dvec, lane_u))
        ub = bias_v[pl.ds(0, LANES)]
        ib = bias_v[pl.ds(LANES, LANES)]

        copies = fetch_blocks(ie_hbm, ri)
        for c in copies:
            c.wait()
        acc = ub + ib + off
        for d in range(DIM):
            dvec = jnp.full((LANES,), d, jnp.int32)
            acc = acc + uc_v[d, pl.ds(0, LANES)] * plsc.load_gather(
                blk_v, (ivec, dvec, lane_i))

        out_v[pl.ds(gbase, LANES)] = 5.5 / (1.0 + jnp.exp(-acc))
        return carry

    lax.fori_loop(0, NGROUPS, group, 0)
    pltpu.sync_copy(out_v, out_hbm.at[pl.ds(base, CHUNK)])


@jax.jit
def kernel(user, item, user_emb, item_emb, user_bias, item_bias, offset):
    run = functools.partial(
        pl.kernel,
        out_type=jax.ShapeDtypeStruct((BATCH,), jnp.float32),
        mesh=plsc.VectorSubcoreMesh(core_axis_name="c", subcore_axis_name="s"),
        scratch_types=[
            pltpu.VMEM((CHUNK,), jnp.int32),
            pltpu.VMEM((CHUNK,), jnp.int32),
            pltpu.VMEM((LANES, DIM, 128), jnp.float32),
            pltpu.VMEM((DIM, LANES), jnp.float32),
            pltpu.VMEM((2 * LANES,), jnp.float32),
            pltpu.VMEM((CHUNK,), jnp.float32),
            pltpu.VMEM((LANES,), jnp.float32),
            pltpu.SemaphoreType.DMA,
        ],
        compiler_params=pltpu.CompilerParams(
            needs_layout_passes=False, use_tc_tiling_on_sc=True),
    )(_body)
    return run(user.astype(jnp.int32), item.astype(jnp.int32),
               user_emb.T, item_emb.T, user_bias, item_bias, offset)
